# baseline (speedup 1.0000x reference)
import jax
import jax.numpy as jnp
from jax.experimental import pallas as pl
from jax.experimental.pallas import tpu as pltpu


def _add_body(ts_ref, x_ref, emb_ref, o_ref):
    o_ref[...] = x_ref[...] + emb_ref[...]


def kernel(x, timestep, embed_table):
    B, S, D = x.shape
    BS = 512
    ts = timestep.astype(jnp.int32)
    table3 = embed_table.reshape(embed_table.shape[0], 1, D)
    return pl.pallas_call(
        _add_body,
        grid_spec=pltpu.PrefetchScalarGridSpec(
            num_scalar_prefetch=1,
            grid=(B, S // BS),
            in_specs=[
                pl.BlockSpec((1, BS, D), lambda b, s, ts_ref: (b, s, 0)),
                pl.BlockSpec((1, 1, D), lambda b, s, ts_ref: (ts_ref[b], 0, 0)),
            ],
            out_specs=pl.BlockSpec((1, BS, D), lambda b, s, ts_ref: (b, s, 0)),
        ),
        out_shape=jax.ShapeDtypeStruct((B, S, D), x.dtype),
    )(ts, x, table3)


# speedup vs baseline: 1.1060x; 1.1060x over previous
import jax
import jax.numpy as jnp
from jax.experimental import pallas as pl
from jax.experimental.pallas import tpu as pltpu


def _add_body(ts_ref, x_ref, emb_ref, o_ref):
    o_ref[...] = x_ref[...] + emb_ref[...]


def kernel(x, timestep, embed_table):
    B, S, D = x.shape
    BS = 2048
    ts = timestep.astype(jnp.int32)
    table3 = embed_table.reshape(embed_table.shape[0], 1, D)
    return pl.pallas_call(
        _add_body,
        grid_spec=pltpu.PrefetchScalarGridSpec(
            num_scalar_prefetch=1,
            grid=(B, S // BS),
            in_specs=[
                pl.BlockSpec((1, BS, D), lambda b, s, ts_ref: (b, s, 0)),
                pl.BlockSpec((1, 1, D), lambda b, s, ts_ref: (ts_ref[b], 0, 0)),
            ],
            out_specs=pl.BlockSpec((1, BS, D), lambda b, s, ts_ref: (b, s, 0)),
        ),
        out_shape=jax.ShapeDtypeStruct((B, S, D), x.dtype),
    )(ts, x, table3)
